# baseline (speedup 1.0000x reference)
import functools
import math

import jax
import jax.numpy as jnp
from jax import lax
from jax.experimental import pallas as pl
from jax.experimental.pallas import tpu as pltpu
from jax.experimental.pallas import tpu_sc as plsc

_DIM = 96
_HEADS = 4
_HD = _DIM // _HEADS
_WS = 4
_TOPK = 2
_HIDDEN = 2 * _DIM
_BN_EPS = 1e-5
_LA_EPS = 1e-6


def _pick(total, target):
    for d in range(min(total, target), 0, -1):
        if total % d == 0:
            return d
    return 1


def _bn_fold(p):
    g, b, m, v = p
    s = g / jnp.sqrt(v + _BN_EPS)
    return s, b - m * s


def _fold(w, in_scale=None, in_shift=None, out_bn=None):
    o = w.shape[0]
    b = jnp.zeros((o,), jnp.float32)
    if in_scale is not None:
        b = b + w @ in_shift
        w = w * in_scale[None, :]
    if out_bn is not None:
        s, sh = _bn_fold(out_bn)
        w = w * s[:, None]
        b = b * s + sh
    return w.T, b


_MM_NB = 1024


def _mm_body(mode, x_ref, *refs):
    if mode in ('pa_gate', 'attn_gate', 'residual'):
        aux_ref = refs[0]
        refs = refs[1:]
    if mode == 'pa_gate':
        w1_ref, w2_ref, b_ref, o_ref = refs
        xb = x_ref[0]
        yb = aux_ref[0]
        acc = (jnp.dot(w1_ref[...], xb, preferred_element_type=jnp.float32)
               + jnp.dot(w2_ref[...], yb, preferred_element_type=jnp.float32)
               + b_ref[...])
        o_ref[0] = xb + jax.nn.sigmoid(acc) * yb
        return
    w_ref, b_ref, o_ref = refs
    acc = jnp.dot(w_ref[...], x_ref[0],
                  preferred_element_type=jnp.float32) + b_ref[...]
    if mode == 'silu':
        o_ref[0] = acc * jax.nn.sigmoid(acc)
    elif mode == 'attn_gate':
        o_ref[0] = aux_ref[0] + x_ref[0] * jax.nn.sigmoid(acc)
    elif mode == 'residual':
        o_ref[0] = aux_ref[0] + acc
    else:
        o_ref[0] = acc


def _mm(x, w, b, mode='none', aux=None):
    bb, ci, m = x.shape
    co = w.shape[0] if mode != 'pa_gate' else w[0].shape[0]
    nb = _pick(m, _MM_NB)
    xspec = pl.BlockSpec((1, ci, nb), lambda bi, i: (bi, 0, i))
    bspec = pl.BlockSpec((co, 1), lambda bi, i: (0, 0))
    ospec = pl.BlockSpec((1, co, nb), lambda bi, i: (bi, 0, i))
    if mode == 'pa_gate':
        w1, w2 = w
        in_specs = [xspec, ospec,
                    pl.BlockSpec(w1.shape, lambda bi, i: (0, 0)),
                    pl.BlockSpec(w2.shape, lambda bi, i: (0, 0)), bspec]
        args = (x, aux, w1, w2, b.reshape(co, 1))
    else:
        wspec = pl.BlockSpec(w.shape, lambda bi, i: (0, 0))
        if aux is not None:
            in_specs = [xspec,
                        pl.BlockSpec((1, aux.shape[1], nb),
                                     lambda bi, i: (bi, 0, i)),
                        wspec, bspec]
            args = (x, aux, w, b.reshape(co, 1))
        else:
            in_specs = [xspec, wspec, bspec]
            args = (x, w, b.reshape(co, 1))
    return pl.pallas_call(
        functools.partial(_mm_body, mode),
        grid=(bb, m // nb),
        in_specs=in_specs,
        out_specs=ospec,
        out_shape=jax.ShapeDtypeStruct((bb, co, m), jnp.float32),
    )(*args)


def _dwconv(x, wgt):
    b, c, h, w = x.shape
    k = wgt.shape[0]
    p = k // 2
    hb = _pick(h, 16)
    g = h // hb
    cb = _pick(c, 32)
    wp = w + 2 * p
    rows = (g + 1) * hb
    xp = jnp.zeros((b, c, rows, wp), x.dtype)
    xp = lax.dynamic_update_slice(
        xp, jnp.pad(x, ((0, 0), (0, 0), (0, 0), (p, p))), (0, 0, p, 0))
    wf = wgt.reshape(k * k, c).T

    def body(a_ref, b_ref, w_ref, o_ref):
        x2 = jnp.concatenate([a_ref[0], b_ref[0]], axis=1)
        wall = w_ref[...]
        acc = jnp.zeros((cb, hb, w), jnp.float32)
        for dy in range(k):
            for dx in range(k):
                acc = acc + (x2[:, dy:dy + hb, dx:dx + w]
                             * wall[:, dy * k + dx][:, None, None])
        o_ref[0] = acc

    return pl.pallas_call(
        body,
        grid=(b, c // cb, g),
        in_specs=[
            pl.BlockSpec((1, cb, hb, wp), lambda bi, ci, gi: (bi, ci, gi, 0)),
            pl.BlockSpec((1, cb, hb, wp), lambda bi, ci, gi: (bi, ci, gi + 1, 0)),
            pl.BlockSpec((cb, k * k), lambda bi, ci, gi: (ci, 0)),
        ],
        out_specs=pl.BlockSpec((1, cb, hb, w),
                               lambda bi, ci, gi: (bi, ci, gi, 0)),
        out_shape=jax.ShapeDtypeStruct((b, c, h, w), jnp.float32),
    )(xp, xp, wf)


def _wmean(xw):
    n, t, c = xw.shape
    nb = _pick(n, 392)

    def body(x_ref, o_ref):
        o_ref[...] = jnp.sum(x_ref[...], axis=1) * (1.0 / t)

    return pl.pallas_call(
        body,
        grid=(n // nb,),
        in_specs=[pl.BlockSpec((nb, t, c), lambda i: (i, 0, 0))],
        out_specs=pl.BlockSpec((nb, c), lambda i: (i, 0)),
        out_shape=jax.ShapeDtypeStruct((n, c), jnp.float32),
    )(xw)


def _route_top2(xd, pd):
    b, nw, c = xd.shape
    scale = 1.0 / math.sqrt(c)
    nb = _pick(nw, 392)

    def body(x_ref, p_ref, i1_ref, i2_ref):
        s = lax.dot_general(x_ref[0], p_ref[0], (((1,), (1,)), ((), ())),
                            preferred_element_type=jnp.float32) * scale
        idx = lax.broadcasted_iota(jnp.int32, (nb, nw), 1)
        m1 = jnp.max(s, axis=1, keepdims=True)
        i1 = jnp.min(jnp.where(s == m1, idx, nw), axis=1)
        s2 = jnp.where(idx == i1[:, None], -jnp.inf, s)
        m2 = jnp.max(s2, axis=1, keepdims=True)
        i2 = jnp.min(jnp.where(s2 == m2, idx, nw), axis=1)
        i1_ref[0, 0] = jnp.broadcast_to(i1[None, :], (8, nb))
        i2_ref[0, 0] = jnp.broadcast_to(i2[None, :], (8, nb))

    nblk = nw // nb
    i1, i2 = pl.pallas_call(
        body,
        grid=(b, nblk),
        in_specs=[pl.BlockSpec((1, nb, c), lambda i, j: (i, j, 0)),
                  pl.BlockSpec((1, nw, c), lambda i, j: (i, 0, 0))],
        out_specs=[pl.BlockSpec((1, 1, 8, nb), lambda i, j: (i, j, 0, 0)),
                   pl.BlockSpec((1, 1, 8, nb), lambda i, j: (i, j, 0, 0))],
        out_shape=[jax.ShapeDtypeStruct((b, nblk, 8, nb), jnp.int32),
                   jax.ShapeDtypeStruct((b, nblk, 8, nb), jnp.int32)],
    )(xd, pd)
    return i1[:, :, 0, :].reshape(b, nw), i2[:, :, 0, :].reshape(b, nw)


def _sc_gather(table, idx):
    v, d = table.shape
    m = idx.shape[0]
    info = plsc.get_sparse_core_info()
    nworkers = info.num_cores * info.num_subcores
    per = m // nworkers
    r = 16
    nch = per // r
    assert per % r == 0 and m % nworkers == 0
    idx3 = idx.reshape(nworkers, nch, r)
    mesh = plsc.VectorSubcoreMesh(core_axis_name="c", subcore_axis_name="s")

    @functools.partial(
        pl.kernel, mesh=mesh,
        out_type=jax.ShapeDtypeStruct((m, d), jnp.float32),
        scratch_types=[
            pltpu.VMEM((nch, r), jnp.int32),
            pltpu.VMEM((r, d), jnp.float32),
            pltpu.SemaphoreType.DMA,
        ],
    )
    def k(t_hbm, i_hbm, o_hbm, idx_v, buf, sem):
        wid = lax.axis_index("s") * info.num_cores + lax.axis_index("c")
        base = wid * per
        pltpu.sync_copy(i_hbm.at[wid], idx_v)

        def step(j, carry):
            pltpu.async_copy(t_hbm.at[idx_v.at[j]], buf, sem).wait()
            pltpu.sync_copy(buf, o_hbm.at[pl.ds(base + j * r, r)])
            return carry

        lax.fori_loop(0, nch, step, 0, unroll=False)

    return k(table, idx3)


def _win_attn(xw, kvg, wq, wproj):
    n, t, c = xw.shape
    kt = kvg.shape[1]
    wb = _pick(n, 64)
    scale = _HD ** -0.5

    def body(x_ref, kv_ref, wq_ref, wp_ref, o_ref):
        xb = x_ref[...]
        q = jnp.dot(xb.reshape(wb * t, c), wq_ref[...],
                    preferred_element_type=jnp.float32).reshape(wb, t, c)
        kv = kv_ref[...]
        outs = []
        for h in range(_HEADS):
            sl = slice(h * _HD, (h + 1) * _HD)
            qh = q[:, :, sl] * scale
            kh = kv[:, :, sl]
            vh = kv[:, :, c + h * _HD:c + (h + 1) * _HD]
            s = lax.dot_general(qh, kh, (((2,), (2,)), ((0,), (0,))),
                                preferred_element_type=jnp.float32)
            s = s - jnp.max(s, axis=-1, keepdims=True)
            e = jnp.exp(s)
            a = e / jnp.sum(e, axis=-1, keepdims=True)
            outs.append(lax.dot_general(a, vh, (((2,), (1,)), ((0,), (0,))),
                                        preferred_element_type=jnp.float32))
        o = jnp.concatenate(outs, axis=-1).reshape(wb * t, c)
        o_ref[...] = jnp.dot(o, wp_ref[...],
                             preferred_element_type=jnp.float32).reshape(wb, t, c)

    return pl.pallas_call(
        body,
        grid=(n // wb,),
        in_specs=[
            pl.BlockSpec((wb, t, c), lambda i: (i, 0, 0)),
            pl.BlockSpec((wb, kt, 2 * c), lambda i: (i, 0, 0)),
            pl.BlockSpec((c, c), lambda i: (0, 0)),
            pl.BlockSpec((c, c), lambda i: (0, 0)),
        ],
        out_specs=pl.BlockSpec((wb, t, c), lambda i: (i, 0, 0)),
        out_shape=jax.ShapeDtypeStruct((n, t, c), jnp.float32),
    )(xw, kvg, wq, wproj)


def _la_phase_a(qkv, b, hw):
    c = _DIM
    tb = _pick(hw, 2048)
    tsteps = hw // tb

    def body(x_ref, o_ref):
        blk = x_ref[0]
        kk = jax.nn.relu(blk[c:2 * c, :])
        vv = jnp.concatenate([blk[2 * c:, :], jnp.ones((c, tb), jnp.float32)],
                             axis=0)
        z = lax.dot_general(kk, vv, (((1,), (1,)), ((), ())),
                            preferred_element_type=jnp.float32)
        ti = pl.program_id(1)

        @pl.when(ti == 0)
        def _():
            o_ref[0] = z

        @pl.when(ti != 0)
        def _():
            o_ref[0] = o_ref[0] + z

    return pl.pallas_call(
        body,
        grid=(b, tsteps),
        in_specs=[pl.BlockSpec((1, 3 * c, tb), lambda bi, ti: (bi, 0, ti))],
        out_specs=pl.BlockSpec((1, c, 2 * c), lambda bi, ti: (bi, 0, 0)),
        out_shape=jax.ShapeDtypeStruct((b, c, 2 * c), jnp.float32),
    )(qkv)


def _la_phase_b(qkv, y1, z, mask, wpj, bpj, b, hw):
    c = _DIM
    tb = _pick(hw, 2048)
    tsteps = hw // tb

    def body(x_ref, y_ref, z_ref, m_ref, w_ref, b_ref, o_ref):
        q = jax.nn.relu(x_ref[0, :c, :])
        zb = z_ref[0]
        mm = m_ref[...]
        kv = zb[:, :c] * mm
        ks = zb[:, c:] * mm
        o = lax.dot_general(kv, q, (((0,), (0,)), ((), ())),
                            preferred_element_type=jnp.float32)
        nrm = jnp.maximum(
            lax.dot_general(ks, q, (((0,), (0,)), ((), ())),
                            preferred_element_type=jnp.float32), _LA_EPS)
        o_ref[0] = (y_ref[0]
                    + lax.dot_general(w_ref[...], o / nrm,
                                      (((0,), (0,)), ((), ())),
                                      preferred_element_type=jnp.float32)
                    + b_ref[...])

    return pl.pallas_call(
        body,
        grid=(b, tsteps),
        in_specs=[
            pl.BlockSpec((1, 3 * c, tb), lambda bi, ti: (bi, 0, ti)),
            pl.BlockSpec((1, c, tb), lambda bi, ti: (bi, 0, ti)),
            pl.BlockSpec((1, c, 2 * c), lambda bi, ti: (bi, 0, 0)),
            pl.BlockSpec((c, c), lambda bi, ti: (0, 0)),
            pl.BlockSpec((c, c), lambda bi, ti: (0, 0)),
            pl.BlockSpec((c, 1), lambda bi, ti: (0, 0)),
        ],
        out_specs=pl.BlockSpec((1, c, tb), lambda bi, ti: (bi, 0, ti)),
        out_shape=jax.ShapeDtypeStruct((b, c, hw), jnp.float32),
    )(qkv, y1, z, mask, wpj, bpj.reshape(c, 1))


def kernel(x, prompt, params):
    p = params
    b, c, h, w = x.shape
    nside = h // _WS
    nw = nside * nside
    t = _WS * _WS
    hw = h * w
    n = b * hw

    def win(z):
        cc = z.shape[1]
        z = z.reshape(b, cc, nside, _WS, nside, _WS)
        z = jnp.transpose(z, (0, 2, 4, 3, 5, 1))
        return z.reshape(b * nw, t, cc)

    def unwin(z):
        cc = z.shape[-1]
        z = z.reshape(b, nside, nside, _WS, _WS, cc)
        z = jnp.transpose(z, (0, 5, 1, 3, 2, 4))
        return z.reshape(b, cc, h, w)

    xw = win(x)
    pw = win(prompt)

    xd = _wmean(xw).reshape(b, nw, c)
    pd = _wmean(pw).reshape(b, nw, c)
    i1, i2 = _route_top2(xd, pd)

    wkv = jnp.concatenate([p['pa_k'], p['pa_v']], axis=0)
    kv_c = _mm(prompt.reshape(b, c, hw), wkv,
               jnp.zeros((2 * c,), jnp.float32))
    kv_rows = win(kv_c.reshape(b, 2 * c, h, w)).reshape(b * nw, t * 2 * c)
    gidx = (jnp.stack([i1, i2], axis=-1)
            + (jnp.arange(b, dtype=jnp.int32) * nw)[:, None, None])
    gathered = _sc_gather(kv_rows, gidx.reshape(-1).astype(jnp.int32))
    kvg = gathered.reshape(b * nw, _TOPK * t, 2 * c)

    wq, _ = _fold(p['pa_q'])
    wpj, _ = _fold(p['pa_proj'])
    aw = _win_attn(xw, kvg, wq, wpj)
    y_c = unwin(aw)

    wg, bg = _fold(p['pa_gate_w'][:, :, 0, 0], out_bn=p['pa_gate_bn'])
    x3 = x.reshape(b, c, hw)
    y1 = _mm(x3, (wg[:c].T, wg[c:].T), bg, mode='pa_gate',
             aux=y_c.reshape(b, c, hw))

    qkv = _mm(y1, p['la_qkv_w'][:, :, 0, 0],
              jnp.zeros((3 * c,), jnp.float32))
    w3w = jnp.transpose(p['la_dw3_w'][:, 0], (1, 2, 0))
    w5w = jnp.transpose(p['la_dw5_w'][:, 0], (1, 2, 0))
    w5c = 0.5 * w5w
    w5c = w5c.at[1:4, 1:4].add(0.5 * w3w)
    w5c = w5c.at[2, 2].add(1.0)
    qkv = _dwconv(qkv.reshape(b, 3 * c, h, w), w5c).reshape(b, 3 * c, hw)

    zmat = _la_phase_a(qkv, b, hw)
    head_ids = jnp.arange(c, dtype=jnp.int32) // _HD
    mask = (head_ids[:, None] == head_ids[None, :]).astype(jnp.float32)
    wlp, blp = _fold(p['la_proj_w'][:, :, 0, 0], out_bn=p['la_proj_bn'])
    y2 = _la_phase_b(qkv, y1, zmat, mask, wlp, blp, b, hw)

    s0, sh0 = _bn_fold(p['ca_norm_bn'])
    wexp, bexp = _fold(p['ca_expand_w'][:, :, 0, 0], in_scale=s0,
                       in_shift=sh0, out_bn=p['ca_expand_bn'])
    ye = _mm(y2, wexp.T, bexp, mode='silu')
    c3 = jnp.transpose(p['ca_dw3_w'][:, 0], (1, 2, 0))
    c5 = jnp.transpose(p['ca_dw5_w'][:, 0], (1, 2, 0))
    c7 = jnp.transpose(p['ca_dw7_w'][:, 0], (1, 2, 0))
    w7c = c7
    w7c = w7c.at[1:6, 1:6].add(c5)
    w7c = w7c.at[2:5, 2:5].add(c3)
    ctx = _dwconv(ye.reshape(b, _HIDDEN, h, w), w7c).reshape(b, _HIDDEN, hw)
    y3 = _mm(ctx, p['ca_attn_w'][:, :, 0, 0],
             jnp.zeros((_HIDDEN,), jnp.float32), mode='attn_gate', aux=ye)
    wpr, bpr = _fold(p['ca_proj_w'][:, :, 0, 0], out_bn=p['ca_proj_bn'])
    y4 = _mm(y3, wpr.T, bpr, mode='residual', aux=y2)

    wf1, bf1 = _fold(p['ffn1_w'][:, :, 0, 0], out_bn=p['ffn1_bn'])
    f1 = _mm(y4, wf1.T, bf1, mode='silu')
    wf2, bf2 = _fold(p['ffn2_w'][:, :, 0, 0], out_bn=p['ffn2_bn'])
    y5 = _mm(f1, wf2.T, bf2, mode='residual', aux=y4)

    return y5.reshape(b, c, h, w)


# speedup vs baseline: 1.5245x; 1.5245x over previous
import functools
import math

import jax
import jax.numpy as jnp
from jax import lax
from jax.experimental import pallas as pl
from jax.experimental.pallas import tpu as pltpu
from jax.experimental.pallas import tpu_sc as plsc

_DIM = 96
_HEADS = 4
_HD = _DIM // _HEADS
_WS = 4
_TOPK = 2
_HIDDEN = 2 * _DIM
_BN_EPS = 1e-5
_LA_EPS = 1e-6


def _pick(total, target):
    for d in range(min(total, target), 0, -1):
        if total % d == 0:
            return d
    return 1


def _bn_fold(p):
    g, b, m, v = p
    s = g / jnp.sqrt(v + _BN_EPS)
    return s, b - m * s


def _fold(w, in_scale=None, in_shift=None, out_bn=None):
    o = w.shape[0]
    b = jnp.zeros((o,), jnp.float32)
    if in_scale is not None:
        b = b + w @ in_shift
        w = w * in_scale[None, :]
    if out_bn is not None:
        s, sh = _bn_fold(out_bn)
        w = w * s[:, None]
        b = b * s + sh
    return w.T, b


_MM_NB = 1024


def _mm_body(mode, x_ref, *refs):
    if mode in ('pa_gate', 'attn_gate', 'residual'):
        aux_ref = refs[0]
        refs = refs[1:]
    if mode == 'pa_gate':
        w1_ref, w2_ref, b_ref, o_ref = refs
        xb = x_ref[...]
        yb = aux_ref[...]
        acc = (jnp.dot(xb, w1_ref[...], preferred_element_type=jnp.float32)
               + jnp.dot(yb, w2_ref[...], preferred_element_type=jnp.float32)
               + b_ref[...])
        o_ref[...] = xb + jax.nn.sigmoid(acc) * yb
        return
    w_ref, b_ref, o_ref = refs
    acc = jnp.dot(x_ref[...], w_ref[...],
                  preferred_element_type=jnp.float32) + b_ref[...]
    if mode == 'silu':
        o_ref[...] = acc * jax.nn.sigmoid(acc)
    elif mode == 'attn_gate':
        o_ref[...] = aux_ref[...] + x_ref[...] * jax.nn.sigmoid(acc)
    elif mode == 'residual':
        o_ref[...] = aux_ref[...] + acc
    else:
        o_ref[...] = acc


def _mm(x, wt, b, mode='none', aux=None):
    n, ci = x.shape
    co = wt.shape[-1] if mode != 'pa_gate' else wt[0].shape[-1]
    nb = _pick(n, _MM_NB)
    xspec = pl.BlockSpec((nb, ci), lambda i: (i, 0))
    bspec = pl.BlockSpec((1, co), lambda i: (0, 0))
    ospec = pl.BlockSpec((nb, co), lambda i: (i, 0))
    if mode == 'pa_gate':
        w1, w2 = wt
        in_specs = [xspec, ospec,
                    pl.BlockSpec(w1.shape, lambda i: (0, 0)),
                    pl.BlockSpec(w2.shape, lambda i: (0, 0)), bspec]
        args = (x, aux, w1, w2, b.reshape(1, co))
    else:
        wspec = pl.BlockSpec(wt.shape, lambda i: (0, 0))
        if aux is not None:
            in_specs = [xspec, pl.BlockSpec((nb, co), lambda i: (i, 0)),
                        wspec, bspec]
            args = (x, aux, wt, b.reshape(1, co))
        else:
            in_specs = [xspec, wspec, bspec]
            args = (x, wt, b.reshape(1, co))
    return pl.pallas_call(
        functools.partial(_mm_body, mode),
        grid=(n // nb,),
        in_specs=in_specs,
        out_specs=ospec,
        out_shape=jax.ShapeDtypeStruct((n, co), jnp.float32),
    )(*args)


def _dwconv(x, wgt):
    b, h, w, c = x.shape
    k = wgt.shape[0]
    p = k // 2
    hb = _pick(h, 16)
    g = h // hb
    wp = w + 2 * p
    rows = (g + 1) * hb
    xp = jnp.zeros((b, rows, wp, c), x.dtype)
    xp = lax.dynamic_update_slice(
        xp, jnp.pad(x, ((0, 0), (0, 0), (p, p), (0, 0))), (0, p, 0, 0))
    wf = wgt.reshape(k * k, c)

    def body(a_ref, b_ref, w_ref, o_ref):
        x2 = jnp.concatenate([a_ref[0], b_ref[0]], axis=0)
        wall = w_ref[...]
        acc = jnp.zeros((hb, w, c), jnp.float32)
        for dy in range(k):
            for dx in range(k):
                acc = acc + (x2[dy:dy + hb, dx:dx + w, :]
                             * wall[dy * k + dx][None, None, :])
        o_ref[0] = acc

    return pl.pallas_call(
        body,
        grid=(b, g),
        in_specs=[
            pl.BlockSpec((1, hb, wp, c), lambda bi, gi: (bi, gi, 0, 0)),
            pl.BlockSpec((1, hb, wp, c), lambda bi, gi: (bi, gi + 1, 0, 0)),
            pl.BlockSpec((k * k, c), lambda bi, gi: (0, 0)),
        ],
        out_specs=pl.BlockSpec((1, hb, w, c), lambda bi, gi: (bi, gi, 0, 0)),
        out_shape=jax.ShapeDtypeStruct((b, h, w, c), jnp.float32),
    )(xp, xp, wf)


def _wmean(xw):
    n, t, c = xw.shape
    nb = _pick(n, 392)

    def body(x_ref, o_ref):
        o_ref[...] = jnp.sum(x_ref[...], axis=1) * (1.0 / t)

    return pl.pallas_call(
        body,
        grid=(n // nb,),
        in_specs=[pl.BlockSpec((nb, t, c), lambda i: (i, 0, 0))],
        out_specs=pl.BlockSpec((nb, c), lambda i: (i, 0)),
        out_shape=jax.ShapeDtypeStruct((n, c), jnp.float32),
    )(xw)


def _route_top2(xd, pd):
    b, nw, c = xd.shape
    scale = 1.0 / math.sqrt(c)
    nb = _pick(nw, 392)

    def body(x_ref, p_ref, i1_ref, i2_ref):
        s = lax.dot_general(x_ref[0], p_ref[0], (((1,), (1,)), ((), ())),
                            preferred_element_type=jnp.float32) * scale
        idx = lax.broadcasted_iota(jnp.int32, (nb, nw), 1)
        m1 = jnp.max(s, axis=1, keepdims=True)
        i1 = jnp.min(jnp.where(s == m1, idx, nw), axis=1)
        s2 = jnp.where(idx == i1[:, None], -jnp.inf, s)
        m2 = jnp.max(s2, axis=1, keepdims=True)
        i2 = jnp.min(jnp.where(s2 == m2, idx, nw), axis=1)
        i1_ref[0, 0] = jnp.broadcast_to(i1[None, :], (8, nb))
        i2_ref[0, 0] = jnp.broadcast_to(i2[None, :], (8, nb))

    nblk = nw // nb
    i1, i2 = pl.pallas_call(
        body,
        grid=(b, nblk),
        in_specs=[pl.BlockSpec((1, nb, c), lambda i, j: (i, j, 0)),
                  pl.BlockSpec((1, nw, c), lambda i, j: (i, 0, 0))],
        out_specs=[pl.BlockSpec((1, 1, 8, nb), lambda i, j: (i, j, 0, 0)),
                   pl.BlockSpec((1, 1, 8, nb), lambda i, j: (i, j, 0, 0))],
        out_shape=[jax.ShapeDtypeStruct((b, nblk, 8, nb), jnp.int32),
                   jax.ShapeDtypeStruct((b, nblk, 8, nb), jnp.int32)],
    )(xd, pd)
    return i1[:, :, 0, :].reshape(b, nw), i2[:, :, 0, :].reshape(b, nw)


def _sc_gather(table, idx):
    v, d = table.shape
    m = idx.shape[0]
    info = plsc.get_sparse_core_info()
    nworkers = info.num_cores * info.num_subcores
    per = m // nworkers
    r = 16
    nch = per // r
    assert per % r == 0 and m % nworkers == 0
    idx3 = idx.reshape(nworkers, nch, r)
    mesh = plsc.VectorSubcoreMesh(core_axis_name="c", subcore_axis_name="s")

    @functools.partial(
        pl.kernel, mesh=mesh,
        out_type=jax.ShapeDtypeStruct((m, d), jnp.float32),
        scratch_types=[
            pltpu.VMEM((nch, r), jnp.int32),
            pltpu.VMEM((r, d), jnp.float32),
            pltpu.SemaphoreType.DMA,
        ],
    )
    def k(t_hbm, i_hbm, o_hbm, idx_v, buf, sem):
        wid = lax.axis_index("s") * info.num_cores + lax.axis_index("c")
        base = wid * per
        pltpu.sync_copy(i_hbm.at[wid], idx_v)

        def step(j, carry):
            pltpu.async_copy(t_hbm.at[idx_v.at[j]], buf, sem).wait()
            pltpu.sync_copy(buf, o_hbm.at[pl.ds(base + j * r, r)])
            return carry

        lax.fori_loop(0, nch, step, 0, unroll=False)

    return k(table, idx3)


def _win_attn(xw, kvg, wq, wproj):
    n, t, c = xw.shape
    kt = kvg.shape[1]
    wb = _pick(n, 64)
    scale = _HD ** -0.5

    def body(x_ref, kv_ref, wq_ref, wp_ref, o_ref):
        xb = x_ref[...]
        q = jnp.dot(xb.reshape(wb * t, c), wq_ref[...],
                    preferred_element_type=jnp.float32).reshape(wb, t, c)
        kv = kv_ref[...]
        outs = []
        for h in range(_HEADS):
            sl = slice(h * _HD, (h + 1) * _HD)
            qh = q[:, :, sl] * scale
            kh = kv[:, :, sl]
            vh = kv[:, :, c + h * _HD:c + (h + 1) * _HD]
            s = lax.dot_general(qh, kh, (((2,), (2,)), ((0,), (0,))),
                                preferred_element_type=jnp.float32)
            s = s - jnp.max(s, axis=-1, keepdims=True)
            e = jnp.exp(s)
            a = e / jnp.sum(e, axis=-1, keepdims=True)
            outs.append(lax.dot_general(a, vh, (((2,), (1,)), ((0,), (0,))),
                                        preferred_element_type=jnp.float32))
        o = jnp.concatenate(outs, axis=-1).reshape(wb * t, c)
        o_ref[...] = jnp.dot(o, wp_ref[...],
                             preferred_element_type=jnp.float32).reshape(wb, t, c)

    return pl.pallas_call(
        body,
        grid=(n // wb,),
        in_specs=[
            pl.BlockSpec((wb, t, c), lambda i: (i, 0, 0)),
            pl.BlockSpec((wb, kt, 2 * c), lambda i: (i, 0, 0)),
            pl.BlockSpec((c, c), lambda i: (0, 0)),
            pl.BlockSpec((c, c), lambda i: (0, 0)),
        ],
        out_specs=pl.BlockSpec((wb, t, c), lambda i: (i, 0, 0)),
        out_shape=jax.ShapeDtypeStruct((n, t, c), jnp.float32),
    )(xw, kvg, wq, wproj)


def _la_phase_a(qkv, b, hw):
    c = _DIM
    tb = _pick(hw, 2048)
    tsteps = hw // tb

    def body(x_ref, o_ref):
        blk = x_ref[...]
        kk = jax.nn.relu(blk[:, c:2 * c])
        vv = jnp.concatenate([blk[:, 2 * c:], jnp.ones((tb, c), jnp.float32)],
                             axis=1)
        z = lax.dot_general(kk, vv, (((0,), (0,)), ((), ())),
                            preferred_element_type=jnp.float32)
        ti = pl.program_id(1)

        @pl.when(ti == 0)
        def _():
            o_ref[0] = z

        @pl.when(ti != 0)
        def _():
            o_ref[0] = o_ref[0] + z

    return pl.pallas_call(
        body,
        grid=(b, tsteps),
        in_specs=[pl.BlockSpec((tb, 3 * c), lambda bi, ti: (bi * tsteps + ti, 0))],
        out_specs=pl.BlockSpec((1, c, 2 * c), lambda bi, ti: (bi, 0, 0)),
        out_shape=jax.ShapeDtypeStruct((b, c, 2 * c), jnp.float32),
    )(qkv)


def _la_phase_b(qkv, y1, z, mask, wpj, bpj, b, hw):
    c = _DIM
    tb = _pick(hw, 2048)
    tsteps = hw // tb

    def body(x_ref, y_ref, z_ref, m_ref, w_ref, b_ref, o_ref):
        q = jax.nn.relu(x_ref[:, :c])
        zb = z_ref[0]
        mm = m_ref[...]
        kv = zb[:, :c] * mm
        ks = zb[:, c:] * mm
        o = jnp.dot(q, kv, preferred_element_type=jnp.float32)
        nrm = jnp.maximum(jnp.dot(q, ks, preferred_element_type=jnp.float32),
                          _LA_EPS)
        o_ref[...] = (y_ref[...]
                      + jnp.dot(o / nrm, w_ref[...],
                                preferred_element_type=jnp.float32)
                      + b_ref[...])

    return pl.pallas_call(
        body,
        grid=(b, tsteps),
        in_specs=[
            pl.BlockSpec((tb, 3 * c), lambda bi, ti: (bi * tsteps + ti, 0)),
            pl.BlockSpec((tb, c), lambda bi, ti: (bi * tsteps + ti, 0)),
            pl.BlockSpec((1, c, 2 * c), lambda bi, ti: (bi, 0, 0)),
            pl.BlockSpec((c, c), lambda bi, ti: (0, 0)),
            pl.BlockSpec((c, c), lambda bi, ti: (0, 0)),
            pl.BlockSpec((1, c), lambda bi, ti: (0, 0)),
        ],
        out_specs=pl.BlockSpec((tb, c), lambda bi, ti: (bi * tsteps + ti, 0)),
        out_shape=jax.ShapeDtypeStruct((b * hw, c), jnp.float32),
    )(qkv, y1, z, mask, wpj, bpj.reshape(1, c))


def kernel(x, prompt, params):
    p = params
    b, c, h, w = x.shape
    nside = h // _WS
    nw = nside * nside
    t = _WS * _WS
    hw = h * w
    n = b * hw

    x_t = jnp.transpose(x, (0, 2, 3, 1))
    pr_t = jnp.transpose(prompt, (0, 2, 3, 1))

    def win(z):
        cc = z.shape[-1]
        z = z.reshape(b, nside, _WS, nside, _WS, cc)
        z = jnp.transpose(z, (0, 1, 3, 2, 4, 5))
        return z.reshape(b * nw, t, cc)

    def unwin(z):
        cc = z.shape[-1]
        z = z.reshape(b, nside, nside, _WS, _WS, cc)
        z = jnp.transpose(z, (0, 1, 3, 2, 4, 5))
        return z.reshape(b, h, w, cc)

    xw = win(x_t)
    pw = win(pr_t)

    xd = _wmean(xw).reshape(b, nw, c)
    pd = _wmean(pw).reshape(b, nw, c)
    i1, i2 = _route_top2(xd, pd)

    wk, _ = _fold(p['pa_k'])
    wv, _ = _fold(p['pa_v'])
    wkv = jnp.concatenate([wk, wv], axis=1)
    kv_t = _mm(pr_t.reshape(n, c), wkv, jnp.zeros((2 * c,), jnp.float32))
    kv_rows = win(kv_t.reshape(b, h, w, 2 * c)).reshape(b * nw, t * 2 * c)
    gidx = (jnp.stack([i1, i2], axis=-1)
            + (jnp.arange(b, dtype=jnp.int32) * nw)[:, None, None])
    gathered = _sc_gather(kv_rows, gidx.reshape(-1).astype(jnp.int32))
    kvg = gathered.reshape(b * nw, _TOPK * t, 2 * c)

    wq, _ = _fold(p['pa_q'])
    wpj, _ = _fold(p['pa_proj'])
    aw = _win_attn(xw, kvg, wq, wpj)
    y_t = unwin(aw)

    wg, bg = _fold(p['pa_gate_w'][:, :, 0, 0], out_bn=p['pa_gate_bn'])
    y1 = _mm(x_t.reshape(n, c), (wg[:c], wg[c:]), bg, mode='pa_gate',
             aux=y_t.reshape(n, c))

    wqkv, _ = _fold(p['la_qkv_w'][:, :, 0, 0])
    qkv = _mm(y1, wqkv, jnp.zeros((3 * c,), jnp.float32))
    w3 = jnp.transpose(p['la_dw3_w'][:, 0], (1, 2, 0))
    w5 = jnp.transpose(p['la_dw5_w'][:, 0], (1, 2, 0))
    w5c = 0.5 * w5
    w5c = w5c.at[1:4, 1:4].add(0.5 * w3)
    w5c = w5c.at[2, 2].add(1.0)
    qkv = _dwconv(qkv.reshape(b, h, w, 3 * c), w5c).reshape(n, 3 * c)

    zmat = _la_phase_a(qkv, b, hw)
    head_ids = jnp.arange(c, dtype=jnp.int32) // _HD
    mask = (head_ids[:, None] == head_ids[None, :]).astype(jnp.float32)
    wlp, blp = _fold(p['la_proj_w'][:, :, 0, 0], out_bn=p['la_proj_bn'])
    y2 = _la_phase_b(qkv, y1, zmat, mask, wlp, blp, b, hw)

    s0, sh0 = _bn_fold(p['ca_norm_bn'])
    wexp, bexp = _fold(p['ca_expand_w'][:, :, 0, 0], in_scale=s0,
                       in_shift=sh0, out_bn=p['ca_expand_bn'])
    ye = _mm(y2, wexp, bexp, mode='silu')
    c3 = jnp.transpose(p['ca_dw3_w'][:, 0], (1, 2, 0))
    c5 = jnp.transpose(p['ca_dw5_w'][:, 0], (1, 2, 0))
    c7 = jnp.transpose(p['ca_dw7_w'][:, 0], (1, 2, 0))
    w7c = c7
    w7c = w7c.at[1:6, 1:6].add(c5)
    w7c = w7c.at[2:5, 2:5].add(c3)
    ctx = _dwconv(ye.reshape(b, h, w, _HIDDEN), w7c).reshape(n, _HIDDEN)
    wat, bat = _fold(p['ca_attn_w'][:, :, 0, 0])
    y3 = _mm(ctx, wat, bat, mode='attn_gate', aux=ye)
    wpr, bpr = _fold(p['ca_proj_w'][:, :, 0, 0], out_bn=p['ca_proj_bn'])
    y4 = _mm(y3, wpr, bpr, mode='residual', aux=y2)

    wf1, bf1 = _fold(p['ffn1_w'][:, :, 0, 0], out_bn=p['ffn1_bn'])
    f1 = _mm(y4, wf1, bf1, mode='silu')
    wf2, bf2 = _fold(p['ffn2_w'][:, :, 0, 0], out_bn=p['ffn2_bn'])
    y5 = _mm(f1, wf2, bf2, mode='residual', aux=y4)

    return jnp.transpose(y5.reshape(b, h, w, c), (0, 3, 1, 2))


# speedup vs baseline: 1.7125x; 1.1233x over previous
import functools
import math

import jax
import jax.numpy as jnp
from jax import lax
from jax.experimental import pallas as pl
from jax.experimental.pallas import tpu as pltpu
from jax.experimental.pallas import tpu_sc as plsc

_DIM = 96
_HEADS = 4
_HD = _DIM // _HEADS
_WS = 4
_TOPK = 2
_HIDDEN = 2 * _DIM
_BN_EPS = 1e-5
_LA_EPS = 1e-6


def _pick(total, target):
    for d in range(min(total, target), 0, -1):
        if total % d == 0:
            return d
    return 1


def _bn_fold(p):
    g, b, m, v = p
    s = g / jnp.sqrt(v + _BN_EPS)
    return s, b - m * s


def _fold(w, in_scale=None, in_shift=None, out_bn=None):
    o = w.shape[0]
    b = jnp.zeros((o,), jnp.float32)
    if in_scale is not None:
        b = b + w @ in_shift
        w = w * in_scale[None, :]
    if out_bn is not None:
        s, sh = _bn_fold(out_bn)
        w = w * s[:, None]
        b = b * s + sh
    return w.T, b


_MM_NB = 4096


def _mm_body(mode, x_ref, *refs):
    if mode in ('pa_gate', 'attn_gate', 'residual'):
        aux_ref = refs[0]
        refs = refs[1:]
    if mode == 'pa_gate':
        w1_ref, w2_ref, b_ref, o_ref = refs
        xb = x_ref[...]
        yb = aux_ref[...]
        acc = (jnp.dot(xb, w1_ref[...], preferred_element_type=jnp.float32)
               + jnp.dot(yb, w2_ref[...], preferred_element_type=jnp.float32)
               + b_ref[...])
        o_ref[...] = xb + jax.nn.sigmoid(acc) * yb
        return
    w_ref, b_ref, o_ref = refs
    acc = jnp.dot(x_ref[...], w_ref[...],
                  preferred_element_type=jnp.float32) + b_ref[...]
    if mode == 'silu':
        o_ref[...] = acc * jax.nn.sigmoid(acc)
    elif mode == 'attn_gate':
        o_ref[...] = aux_ref[...] + x_ref[...] * jax.nn.sigmoid(acc)
    elif mode == 'residual':
        o_ref[...] = aux_ref[...] + acc
    else:
        o_ref[...] = acc


def _mm(x, wt, b, mode='none', aux=None):
    n, ci = x.shape
    co = wt.shape[-1] if mode != 'pa_gate' else wt[0].shape[-1]
    nb = _pick(n, _MM_NB)
    xspec = pl.BlockSpec((nb, ci), lambda i: (i, 0))
    bspec = pl.BlockSpec((1, co), lambda i: (0, 0))
    ospec = pl.BlockSpec((nb, co), lambda i: (i, 0))
    if mode == 'pa_gate':
        w1, w2 = wt
        in_specs = [xspec, ospec,
                    pl.BlockSpec(w1.shape, lambda i: (0, 0)),
                    pl.BlockSpec(w2.shape, lambda i: (0, 0)), bspec]
        args = (x, aux, w1, w2, b.reshape(1, co))
    else:
        wspec = pl.BlockSpec(wt.shape, lambda i: (0, 0))
        if aux is not None:
            in_specs = [xspec, pl.BlockSpec((nb, co), lambda i: (i, 0)),
                        wspec, bspec]
            args = (x, aux, wt, b.reshape(1, co))
        else:
            in_specs = [xspec, wspec, bspec]
            args = (x, wt, b.reshape(1, co))
    return pl.pallas_call(
        functools.partial(_mm_body, mode),
        grid=(n // nb,),
        in_specs=in_specs,
        out_specs=ospec,
        out_shape=jax.ShapeDtypeStruct((n, co), jnp.float32),
    )(*args)


def _dwconv(x, wgt):
    b, h, w, c = x.shape
    k = wgt.shape[0]
    p = k // 2
    hb = _pick(h, 16)
    g = h // hb
    wp = w + 2 * p
    rows = (g + 1) * hb
    xp = jnp.zeros((b, rows, wp, c), x.dtype)
    xp = lax.dynamic_update_slice(
        xp, jnp.pad(x, ((0, 0), (0, 0), (p, p), (0, 0))), (0, p, 0, 0))
    wf = wgt.reshape(k * k, c)

    def body(a_ref, b_ref, w_ref, o_ref):
        x2 = jnp.concatenate([a_ref[0], b_ref[0]], axis=0)
        wall = w_ref[...]
        acc = jnp.zeros((hb, w, c), jnp.float32)
        for dy in range(k):
            for dx in range(k):
                acc = acc + (x2[dy:dy + hb, dx:dx + w, :]
                             * wall[dy * k + dx][None, None, :])
        o_ref[0] = acc

    return pl.pallas_call(
        body,
        grid=(b, g),
        in_specs=[
            pl.BlockSpec((1, hb, wp, c), lambda bi, gi: (bi, gi, 0, 0)),
            pl.BlockSpec((1, hb, wp, c), lambda bi, gi: (bi, gi + 1, 0, 0)),
            pl.BlockSpec((k * k, c), lambda bi, gi: (0, 0)),
        ],
        out_specs=pl.BlockSpec((1, hb, w, c), lambda bi, gi: (bi, gi, 0, 0)),
        out_shape=jax.ShapeDtypeStruct((b, h, w, c), jnp.float32),
    )(xp, xp, wf)


def _wmean(xw):
    n, t, c = xw.shape
    nb = _pick(n, 784)

    def body(x_ref, o_ref):
        o_ref[...] = jnp.sum(x_ref[...], axis=1) * (1.0 / t)

    return pl.pallas_call(
        body,
        grid=(n // nb,),
        in_specs=[pl.BlockSpec((nb, t, c), lambda i: (i, 0, 0))],
        out_specs=pl.BlockSpec((nb, c), lambda i: (i, 0)),
        out_shape=jax.ShapeDtypeStruct((n, c), jnp.float32),
    )(xw)


def _route_top2(xd, pd):
    b, nw, c = xd.shape
    scale = 1.0 / math.sqrt(c)
    nb = _pick(nw, 392)

    def body(x_ref, p_ref, i1_ref, i2_ref):
        s = lax.dot_general(x_ref[0], p_ref[0], (((1,), (1,)), ((), ())),
                            preferred_element_type=jnp.float32) * scale
        idx = lax.broadcasted_iota(jnp.int32, (nb, nw), 1)
        m1 = jnp.max(s, axis=1, keepdims=True)
        i1 = jnp.min(jnp.where(s == m1, idx, nw), axis=1)
        s2 = jnp.where(idx == i1[:, None], -jnp.inf, s)
        m2 = jnp.max(s2, axis=1, keepdims=True)
        i2 = jnp.min(jnp.where(s2 == m2, idx, nw), axis=1)
        i1_ref[0, 0] = jnp.broadcast_to(i1[None, :], (8, nb))
        i2_ref[0, 0] = jnp.broadcast_to(i2[None, :], (8, nb))

    nblk = nw // nb
    i1, i2 = pl.pallas_call(
        body,
        grid=(b, nblk),
        in_specs=[pl.BlockSpec((1, nb, c), lambda i, j: (i, j, 0)),
                  pl.BlockSpec((1, nw, c), lambda i, j: (i, 0, 0))],
        out_specs=[pl.BlockSpec((1, 1, 8, nb), lambda i, j: (i, j, 0, 0)),
                   pl.BlockSpec((1, 1, 8, nb), lambda i, j: (i, j, 0, 0))],
        out_shape=[jax.ShapeDtypeStruct((b, nblk, 8, nb), jnp.int32),
                   jax.ShapeDtypeStruct((b, nblk, 8, nb), jnp.int32)],
    )(xd, pd)
    return i1[:, :, 0, :].reshape(b, nw), i2[:, :, 0, :].reshape(b, nw)


def _sc_gather(table, idx):
    v, d = table.shape
    m = idx.shape[0]
    info = plsc.get_sparse_core_info()
    nworkers = info.num_cores * info.num_subcores
    per = m // nworkers
    r = 16
    nch = per // r
    assert per % r == 0 and m % nworkers == 0
    idx3 = idx.reshape(nworkers, nch, r)
    mesh = plsc.VectorSubcoreMesh(core_axis_name="c", subcore_axis_name="s")

    @functools.partial(
        pl.kernel, mesh=mesh,
        out_type=jax.ShapeDtypeStruct((m, d), jnp.float32),
        scratch_types=[
            pltpu.VMEM((nch, r), jnp.int32),
            pltpu.VMEM((r, d), jnp.float32),
            pltpu.SemaphoreType.DMA,
        ],
    )
    def k(t_hbm, i_hbm, o_hbm, idx_v, buf, sem):
        wid = lax.axis_index("s") * info.num_cores + lax.axis_index("c")
        base = wid * per
        pltpu.sync_copy(i_hbm.at[wid], idx_v)

        def step(j, carry):
            pltpu.async_copy(t_hbm.at[idx_v.at[j]], buf, sem).wait()
            pltpu.sync_copy(buf, o_hbm.at[pl.ds(base + j * r, r)])
            return carry

        lax.fori_loop(0, nch, step, 0, unroll=False)

    return k(table, idx3)


def _win_attn(xw, kvg, wq, wproj):
    n, t, c = xw.shape
    kt = kvg.shape[1]
    wb = _pick(n, 128)
    scale = _HD ** -0.5

    def body(x_ref, kv_ref, wq_ref, wp_ref, o_ref):
        xb = x_ref[...]
        q = jnp.dot(xb.reshape(wb * t, c), wq_ref[...],
                    preferred_element_type=jnp.float32).reshape(wb, t, c)
        kv = kv_ref[...]
        outs = []
        for h in range(_HEADS):
            sl = slice(h * _HD, (h + 1) * _HD)
            qh = q[:, :, sl] * scale
            kh = kv[:, :, sl]
            vh = kv[:, :, c + h * _HD:c + (h + 1) * _HD]
            s = lax.dot_general(qh, kh, (((2,), (2,)), ((0,), (0,))),
                                preferred_element_type=jnp.float32)
            s = s - jnp.max(s, axis=-1, keepdims=True)
            e = jnp.exp(s)
            a = e / jnp.sum(e, axis=-1, keepdims=True)
            outs.append(lax.dot_general(a, vh, (((2,), (1,)), ((0,), (0,))),
                                        preferred_element_type=jnp.float32))
        o = jnp.concatenate(outs, axis=-1).reshape(wb * t, c)
        o_ref[...] = jnp.dot(o, wp_ref[...],
                             preferred_element_type=jnp.float32).reshape(wb, t, c)

    return pl.pallas_call(
        body,
        grid=(n // wb,),
        in_specs=[
            pl.BlockSpec((wb, t, c), lambda i: (i, 0, 0)),
            pl.BlockSpec((wb, kt, 2 * c), lambda i: (i, 0, 0)),
            pl.BlockSpec((c, c), lambda i: (0, 0)),
            pl.BlockSpec((c, c), lambda i: (0, 0)),
        ],
        out_specs=pl.BlockSpec((wb, t, c), lambda i: (i, 0, 0)),
        out_shape=jax.ShapeDtypeStruct((n, t, c), jnp.float32),
    )(xw, kvg, wq, wproj)


def _la_phase_a(qkv, b, hw):
    c = _DIM
    tb = _pick(hw, 3584)
    tsteps = hw // tb

    def body(x_ref, o_ref):
        blk = x_ref[...]
        kk = jax.nn.relu(blk[:, c:2 * c])
        vv = jnp.concatenate([blk[:, 2 * c:], jnp.ones((tb, c), jnp.float32)],
                             axis=1)
        z = lax.dot_general(kk, vv, (((0,), (0,)), ((), ())),
                            preferred_element_type=jnp.float32)
        ti = pl.program_id(1)

        @pl.when(ti == 0)
        def _():
            o_ref[0] = z

        @pl.when(ti != 0)
        def _():
            o_ref[0] = o_ref[0] + z

    return pl.pallas_call(
        body,
        grid=(b, tsteps),
        in_specs=[pl.BlockSpec((tb, 3 * c), lambda bi, ti: (bi * tsteps + ti, 0))],
        out_specs=pl.BlockSpec((1, c, 2 * c), lambda bi, ti: (bi, 0, 0)),
        out_shape=jax.ShapeDtypeStruct((b, c, 2 * c), jnp.float32),
    )(qkv)


def _la_phase_b(qkv, y1, z, mask, wpj, bpj, b, hw):
    c = _DIM
    tb = _pick(hw, 3584)
    tsteps = hw // tb

    def body(x_ref, y_ref, z_ref, m_ref, w_ref, b_ref, o_ref):
        q = jax.nn.relu(x_ref[:, :c])
        zb = z_ref[0]
        mm = m_ref[...]
        kv = zb[:, :c] * mm
        ks = zb[:, c:] * mm
        o = jnp.dot(q, kv, preferred_element_type=jnp.float32)
        nrm = jnp.maximum(jnp.dot(q, ks, preferred_element_type=jnp.float32),
                          _LA_EPS)
        o_ref[...] = (y_ref[...]
                      + jnp.dot(o / nrm, w_ref[...],
                                preferred_element_type=jnp.float32)
                      + b_ref[...])

    return pl.pallas_call(
        body,
        grid=(b, tsteps),
        in_specs=[
            pl.BlockSpec((tb, 3 * c), lambda bi, ti: (bi * tsteps + ti, 0)),
            pl.BlockSpec((tb, c), lambda bi, ti: (bi * tsteps + ti, 0)),
            pl.BlockSpec((1, c, 2 * c), lambda bi, ti: (bi, 0, 0)),
            pl.BlockSpec((c, c), lambda bi, ti: (0, 0)),
            pl.BlockSpec((c, c), lambda bi, ti: (0, 0)),
            pl.BlockSpec((1, c), lambda bi, ti: (0, 0)),
        ],
        out_specs=pl.BlockSpec((tb, c), lambda bi, ti: (bi * tsteps + ti, 0)),
        out_shape=jax.ShapeDtypeStruct((b * hw, c), jnp.float32),
    )(qkv, y1, z, mask, wpj, bpj.reshape(1, c))


def kernel(x, prompt, params):
    p = params
    b, c, h, w = x.shape
    nside = h // _WS
    nw = nside * nside
    t = _WS * _WS
    hw = h * w
    n = b * hw

    x_t = jnp.transpose(x, (0, 2, 3, 1))
    pr_t = jnp.transpose(prompt, (0, 2, 3, 1))

    def win(z):
        cc = z.shape[-1]
        z = z.reshape(b, nside, _WS, nside, _WS, cc)
        z = jnp.transpose(z, (0, 1, 3, 2, 4, 5))
        return z.reshape(b * nw, t, cc)

    def unwin(z):
        cc = z.shape[-1]
        z = z.reshape(b, nside, nside, _WS, _WS, cc)
        z = jnp.transpose(z, (0, 1, 3, 2, 4, 5))
        return z.reshape(b, h, w, cc)

    xw = win(x_t)
    pw = win(pr_t)

    xd = _wmean(xw).reshape(b, nw, c)
    pd = _wmean(pw).reshape(b, nw, c)
    i1, i2 = _route_top2(xd, pd)

    wk, _ = _fold(p['pa_k'])
    wv, _ = _fold(p['pa_v'])
    wkv = jnp.concatenate([wk, wv], axis=1)
    kv_t = _mm(pr_t.reshape(n, c), wkv, jnp.zeros((2 * c,), jnp.float32))
    kv_rows = win(kv_t.reshape(b, h, w, 2 * c)).reshape(b * nw, t * 2 * c)
    gidx = (jnp.stack([i1, i2], axis=-1)
            + (jnp.arange(b, dtype=jnp.int32) * nw)[:, None, None])
    gathered = _sc_gather(kv_rows, gidx.reshape(-1).astype(jnp.int32))
    kvg = gathered.reshape(b * nw, _TOPK * t, 2 * c)

    wq, _ = _fold(p['pa_q'])
    wpj, _ = _fold(p['pa_proj'])
    aw = _win_attn(xw, kvg, wq, wpj)
    y_t = unwin(aw)

    wg, bg = _fold(p['pa_gate_w'][:, :, 0, 0], out_bn=p['pa_gate_bn'])
    y1 = _mm(x_t.reshape(n, c), (wg[:c], wg[c:]), bg, mode='pa_gate',
             aux=y_t.reshape(n, c))

    wqkv, _ = _fold(p['la_qkv_w'][:, :, 0, 0])
    qkv = _mm(y1, wqkv, jnp.zeros((3 * c,), jnp.float32))
    w3 = jnp.transpose(p['la_dw3_w'][:, 0], (1, 2, 0))
    w5 = jnp.transpose(p['la_dw5_w'][:, 0], (1, 2, 0))
    w5c = 0.5 * w5
    w5c = w5c.at[1:4, 1:4].add(0.5 * w3)
    w5c = w5c.at[2, 2].add(1.0)
    qkv = _dwconv(qkv.reshape(b, h, w, 3 * c), w5c).reshape(n, 3 * c)

    zmat = _la_phase_a(qkv, b, hw)
    head_ids = jnp.arange(c, dtype=jnp.int32) // _HD
    mask = (head_ids[:, None] == head_ids[None, :]).astype(jnp.float32)
    wlp, blp = _fold(p['la_proj_w'][:, :, 0, 0], out_bn=p['la_proj_bn'])
    y2 = _la_phase_b(qkv, y1, zmat, mask, wlp, blp, b, hw)

    s0, sh0 = _bn_fold(p['ca_norm_bn'])
    wexp, bexp = _fold(p['ca_expand_w'][:, :, 0, 0], in_scale=s0,
                       in_shift=sh0, out_bn=p['ca_expand_bn'])
    ye = _mm(y2, wexp, bexp, mode='silu')
    c3 = jnp.transpose(p['ca_dw3_w'][:, 0], (1, 2, 0))
    c5 = jnp.transpose(p['ca_dw5_w'][:, 0], (1, 2, 0))
    c7 = jnp.transpose(p['ca_dw7_w'][:, 0], (1, 2, 0))
    w7c = c7
    w7c = w7c.at[1:6, 1:6].add(c5)
    w7c = w7c.at[2:5, 2:5].add(c3)
    ctx = _dwconv(ye.reshape(b, h, w, _HIDDEN), w7c).reshape(n, _HIDDEN)
    wat, bat = _fold(p['ca_attn_w'][:, :, 0, 0])
    y3 = _mm(ctx, wat, bat, mode='attn_gate', aux=ye)
    wpr, bpr = _fold(p['ca_proj_w'][:, :, 0, 0], out_bn=p['ca_proj_bn'])
    y4 = _mm(y3, wpr, bpr, mode='residual', aux=y2)

    wf1, bf1 = _fold(p['ffn1_w'][:, :, 0, 0], out_bn=p['ffn1_bn'])
    f1 = _mm(y4, wf1, bf1, mode='silu')
    wf2, bf2 = _fold(p['ffn2_w'][:, :, 0, 0], out_bn=p['ffn2_bn'])
    y5 = _mm(f1, wf2, bf2, mode='residual', aux=y4)

    return jnp.transpose(y5.reshape(b, h, w, c), (0, 3, 1, 2))
